# speedup vs baseline: 1.0000x; 1.0000x over previous
import jax
import jax.numpy as jnp
import numpy as np
from jax.experimental import pallas as pl


def _gcn(x, ei, W, b):
    N = x.shape[0]
    xw = x @ W
    loop = jnp.arange(N, dtype=ei.dtype)
    src = jnp.concatenate([ei[0], loop])
    dst = jnp.concatenate([ei[1], loop])
    deg = jax.ops.segment_sum(jnp.ones(src.shape[0], dtype=x.dtype), dst, num_segments=N)
    dinv = jax.lax.rsqrt(jnp.maximum(deg, 1e-12))
    norm = dinv[src] * dinv[dst]
    out = jax.ops.segment_sum(xw[src] * norm[:, None], dst, num_segments=N)
    return out + b


def _mha(X, wq, wk, wv, wo, n_heads=4):
    dk = X.shape[1] // n_heads
    Q = jnp.transpose((X @ wq).reshape(-1, n_heads, dk), (1, 0, 2))
    K = jnp.transpose((X @ wk).reshape(-1, n_heads, dk), (1, 0, 2))
    V = jnp.transpose((X @ wv).reshape(-1, n_heads, dk), (1, 0, 2))
    scores = jnp.matmul(Q, jnp.transpose(K, (0, 2, 1))) / np.sqrt(dk)
    attn = jax.nn.softmax(scores, axis=-1)
    ctx = jnp.matmul(attn, V)
    ctx = jnp.transpose(ctx, (0, 2, 1)).reshape(-1, n_heads * dk)
    return ctx @ wo


def kernel(drug_x, drug_edge_index, drug_batch, drug_smiles,
           pro_x, pro_edge_index, pro_batch, pro_protein, params):
    p = params
    relu = jax.nn.relu
    nb = drug_smiles.shape[0]
    x = relu(_gcn(drug_x, drug_edge_index, p['dc1_w'], p['dc1_b']))
    x = relu(_gcn(x, drug_edge_index, p['dc2_w'], p['dc2_b']))
    x = relu(_gcn(x, drug_edge_index, p['dc3_w'], p['dc3_b']))
    x = jax.ops.segment_max(x, drug_batch, num_segments=nb)
    x = relu(x @ p['dfc1_w'] + p['dfc1_b'])
    x = x @ p['dfc2_w'] + p['dfc2_b']
    px = relu(_gcn(pro_x, pro_edge_index, p['pc1_w'], p['pc1_b']))
    px = relu(_gcn(px, pro_edge_index, p['pc2_w'], p['pc2_b']))
    px = relu(_gcn(px, pro_edge_index, p['pc3_w'], p['pc3_b']))
    px = jax.ops.segment_max(px, pro_batch, num_segments=nb)
    px = relu(px @ p['pfc1_w'] + p['pfc1_b'])
    px = px @ p['pfc2_w'] + p['pfc2_b']
    pe = relu(pro_protein @ p['pe1_w'] + p['pe1_b'])
    pe = relu(pe @ p['pe2_w'] + p['pe2_b'])
    de = relu(drug_smiles @ p['de1_w'] + p['de1_b'])
    de = relu(de @ p['de2_w'] + p['de2_b'])
    xc = jnp.concatenate([x, de, px, pe], axis=1)
    att = _mha(xc, p['wq'], p['wk'], p['wv'], p['wo'])
    xc = jnp.concatenate([xc, att], axis=1)
    xc = relu(xc @ p['fc1_w'] + p['fc1_b'])
    xc = relu(xc @ p['fc2_w'] + p['fc2_b'])
    return xc @ p['out_w'] + p['out_b']


# speedup vs baseline: 2.3136x; 2.3136x over previous
import functools

import jax
import jax.numpy as jnp
import numpy as np
from jax import lax
from jax.experimental import pallas as pl
from jax.experimental.pallas import tpu as pltpu
from jax.experimental.pallas import tpu_sc as plsc

_NC, _NS, _L = 2, 16, 16
_NW = _NC * _NS


def _pad_to(x, m):
    return ((x + m - 1) // m) * m


def _pick_block(epw, max_words=8192):
    best = None
    for nb in range(1, epw + 1):
        if epw % nb:
            continue
        cb = epw // nb
        if cb % 16 == 0 and cb <= max_words:
            best = cb
            break
    if best is None:
        raise ValueError(f"no block for {epw}")
    return best


@functools.lru_cache(maxsize=None)
def _make_degree_kernel(E_pad, N_pad):
    epw = E_pad // _NW
    assert epw % 16 == 0 and epw % 8 == 0
    CB = _pick_block(epw)
    NBLK = epw // CB
    mesh = plsc.VectorSubcoreMesh(core_axis_name="c", subcore_axis_name="s", num_cores=_NC, num_subcores=_NS)

    @functools.partial(
        pl.kernel,
        out_type=jax.ShapeDtypeStruct((_NW, N_pad), jnp.float32),
        mesh=mesh,
        scratch_types=[
            pltpu.VMEM((N_pad,), jnp.float32),
            pltpu.VMEM((CB,), jnp.int32),
        ],
        compiler_params=pltpu.CompilerParams(needs_layout_passes=False),
    )
    def deg_kernel(dst_hbm, out_hbm, cnt, buf):
        c = lax.axis_index("c")
        s = lax.axis_index("s")
        wid = s * _NC + c

        zeros = jnp.zeros((_L,), jnp.float32)

        def zbody(i, _):
            cnt[pl.ds(i * _L, _L)] = zeros
            return 0

        lax.fori_loop(0, N_pad // _L, zbody, 0)

        ones = jnp.ones((_L,), jnp.float32)

        def blk(b, _):
            off = wid * epw + b * CB
            pltpu.sync_copy(dst_hbm.at[pl.ds(off, CB)], buf)

            def vec(j, _):
                idx = buf[pl.ds(j * _L, _L)]
                plsc.addupdate_scatter(cnt, [idx], ones)
                return 0

            lax.fori_loop(0, CB // _L, vec, 0)
            return 0

        lax.fori_loop(0, NBLK, blk, 0)
        pltpu.sync_copy(cnt, out_hbm.at[wid])

    return deg_kernel


_SPMEM_BUDGET = 8_000_000
_K = 128
_FLUSH_AT = _K - 16


def _pick_range(N_pad, F_pad):
    best = None
    units = N_pad // 512
    for k in range(1, units + 1):
        if units % k:
            continue
        if (512 * k + 16) * F_pad * 4 <= _SPMEM_BUDGET:
            best = 512 * k
    return best


@functools.lru_cache(maxsize=None)
def _make_agg_kernel(N_pad, F_pad, E_pad):
    R = _pick_range(N_pad, F_pad)
    NRANGE = N_pad // R
    NPASS = (NRANGE + 1) // 2
    RT = R + 16
    epw = E_pad // _NS
    CB = _pick_block(epw, 8192)
    NBLK = epw // CB
    rows_per_tile = RT // _L
    ZR = 16
    zq, zr = divmod(rows_per_tile, ZR)
    wb_rows = R // _NS

    mesh = plsc.VectorSubcoreMesh(core_axis_name="c", subcore_axis_name="s", num_cores=_NC, num_subcores=_NS)

    @functools.partial(
        pl.kernel,
        out_type=jax.ShapeDtypeStruct((N_pad, F_pad), jnp.float32),
        mesh=mesh,
        scratch_types=[
            pltpu.VMEM_SHARED((RT, F_pad), jnp.float32),
            pltpu.VMEM((ZR, F_pad), jnp.float32),
            pltpu.VMEM((CB,), jnp.int32),
            pltpu.VMEM((CB,), jnp.int32),
            pltpu.VMEM((_K,), jnp.int32),
            pltpu.VMEM((_K,), jnp.int32),
            pltpu.VMEM((_K, F_pad), jnp.float32),
            pltpu.SemaphoreType.DMA,
        ],
        compiler_params=pltpu.CompilerParams(
            needs_layout_passes=False, use_tc_tiling_on_sc=False),
    )
    def agg_kernel(src_hbm, dst_hbm, y_hbm, out_hbm,
                   acc, zbuf, dbuf, sbuf, sidx, lidx, rows, sem):
        c = lax.axis_index("c")
        s = lax.axis_index("s")
        iota = lax.iota(jnp.int32, _L)
        zeros_f = jnp.zeros((_L,), jnp.float32)
        zeros_i = jnp.zeros((_L,), jnp.int32)
        trash_i = jnp.full((_L,), R, jnp.int32)

        def zb(i, _):
            def zc(k, _):
                zbuf[i, pl.ds(k * _L, _L)] = zeros_f
                return 0
            lax.fori_loop(0, F_pad // _L, zc, 0)
            return 0
        lax.fori_loop(0, ZR, zb, 0)

        def flush(f):
            pad = f + iota
            pm = pad < _K
            plsc.store_scatter(sidx, [pad], zeros_i, mask=pm)
            plsc.store_scatter(lidx, [pad], trash_i, mask=pm)
            pltpu.async_copy(y_hbm.at[sidx], rows, sem).wait()
            pltpu.sync_copy(rows, acc.at[lidx], add=True)
            return jnp.int32(0)

        def one_pass(p, _):
            r_idx = p * _NC + c
            base = r_idx * R

            @pl.when(r_idx < NRANGE)
            def _():
                row0 = s * rows_per_tile

                def zz(qq, _):
                    pltpu.sync_copy(zbuf, acc.at[pl.ds(row0 + qq * ZR, ZR)])
                    return 0
                lax.fori_loop(0, zq, zz, 0)
                if zr:
                    pltpu.sync_copy(zbuf.at[pl.ds(0, zr)],
                                    acc.at[pl.ds(row0 + zq * ZR, zr)])
                plsc.subcore_barrier()

                def blk(b, fill):
                    off = s * epw + b * CB
                    pltpu.sync_copy(dst_hbm.at[pl.ds(off, CB)], dbuf)
                    pltpu.sync_copy(src_hbm.at[pl.ds(off, CB)], sbuf)

                    def vec(j, fill):
                        d = dbuf[pl.ds(j * _L, _L)]
                        sv = sbuf[pl.ds(j * _L, _L)]
                        m = (d >= base) & (d < base + R)
                        loc = d - base
                        plsc.store_compressed(sidx.at[pl.ds(fill, _L)],
                                              sv, mask=m)
                        plsc.store_compressed(lidx.at[pl.ds(fill, _L)],
                                              loc, mask=m)
                        cnt = jnp.max(plsc.all_reduce_population_count(m))
                        f2 = fill + cnt
                        return lax.cond(f2 >= _FLUSH_AT, flush,
                                        lambda f: f, f2)

                    return lax.fori_loop(0, CB // _L, vec, fill)

                fill = lax.fori_loop(0, NBLK, blk, jnp.int32(0))

                def padt(t, _):
                    pad = fill + iota + t * _L
                    pm = pad < _K
                    plsc.store_scatter(sidx, [pad], zeros_i, mask=pm)
                    plsc.store_scatter(lidx, [pad], trash_i, mask=pm)
                    return 0
                lax.fori_loop(0, _K // _L, padt, 0)
                pltpu.async_copy(y_hbm.at[sidx], rows, sem).wait()
                pltpu.sync_copy(rows, acc.at[lidx], add=True)

                plsc.subcore_barrier()
                pltpu.sync_copy(
                    acc.at[pl.ds(s * wb_rows, wb_rows)],
                    out_hbm.at[pl.ds(base + s * wb_rows, wb_rows)])
                plsc.subcore_barrier()
            return 0

        lax.fori_loop(0, NPASS, one_pass, 0)

    return agg_kernel


def _sc_aggregate(y, src, dst, N, F):
    E = src.shape[0]
    N_pad = _pad_to(N + _L, 512)
    F_pad = _pad_to(F, _L)
    E_pad = _pad_to(E, 256)
    ypad = jnp.zeros((N_pad, F_pad), jnp.float32).at[:N, :F].set(y)
    if E_pad != E:
        src = jnp.concatenate(
            [src, jnp.zeros((E_pad - E,), jnp.int32)])
        dst = jnp.concatenate(
            [dst, jnp.full((E_pad - E,), N, jnp.int32)])
    agg = _make_agg_kernel(N_pad, F_pad, E_pad)(
        src.astype(jnp.int32), dst.astype(jnp.int32), ypad)
    return agg[:N, :F]


def _sc_degree(dst, N):
    E = dst.shape[0]
    E_pad = _pad_to(E, 512)
    N_pad = _pad_to(N + _L, 512)
    if E_pad != E:
        dst = jnp.concatenate(
            [dst, jnp.full((E_pad - E,), N, dtype=jnp.int32)])
    partial = _make_degree_kernel(E_pad, N_pad)(dst.astype(jnp.int32))
    return partial.sum(axis=0)[:N]


def _gcn(x, ei, W, b, dinv):
    xw = x @ W
    y = dinv[:, None] * xw
    N, F = y.shape
    agg = _sc_aggregate(y, ei[0], ei[1], N, F)
    return dinv[:, None] * (agg + y) + b


def _mha(X, wq, wk, wv, wo, n_heads=4):
    dk = X.shape[1] // n_heads
    Q = jnp.transpose((X @ wq).reshape(-1, n_heads, dk), (1, 0, 2))
    K = jnp.transpose((X @ wk).reshape(-1, n_heads, dk), (1, 0, 2))
    V = jnp.transpose((X @ wv).reshape(-1, n_heads, dk), (1, 0, 2))
    scores = jnp.matmul(Q, jnp.transpose(K, (0, 2, 1))) / np.sqrt(dk)
    attn = jax.nn.softmax(scores, axis=-1)
    ctx = jnp.matmul(attn, V)
    ctx = jnp.transpose(ctx, (0, 2, 1)).reshape(-1, n_heads * dk)
    return ctx @ wo


def kernel(drug_x, drug_edge_index, drug_batch, drug_smiles,
           pro_x, pro_edge_index, pro_batch, pro_protein, params):
    p = params
    relu = jax.nn.relu
    nb = drug_smiles.shape[0]

    d_deg = _sc_degree(drug_edge_index[1], drug_x.shape[0]) + 1.0
    d_dinv = lax.rsqrt(jnp.maximum(d_deg, 1e-12))
    p_deg = _sc_degree(pro_edge_index[1], pro_x.shape[0]) + 1.0
    p_dinv = lax.rsqrt(jnp.maximum(p_deg, 1e-12))

    x = relu(_gcn(drug_x, drug_edge_index, p['dc1_w'], p['dc1_b'], d_dinv))
    x = relu(_gcn(x, drug_edge_index, p['dc2_w'], p['dc2_b'], d_dinv))
    x = relu(_gcn(x, drug_edge_index, p['dc3_w'], p['dc3_b'], d_dinv))
    x = jax.ops.segment_max(x, drug_batch, num_segments=nb)
    x = relu(x @ p['dfc1_w'] + p['dfc1_b'])
    x = x @ p['dfc2_w'] + p['dfc2_b']
    px = relu(_gcn(pro_x, pro_edge_index, p['pc1_w'], p['pc1_b'], p_dinv))
    px = relu(_gcn(px, pro_edge_index, p['pc2_w'], p['pc2_b'], p_dinv))
    px = relu(_gcn(px, pro_edge_index, p['pc3_w'], p['pc3_b'], p_dinv))
    px = jax.ops.segment_max(px, pro_batch, num_segments=nb)
    px = relu(px @ p['pfc1_w'] + p['pfc1_b'])
    px = px @ p['pfc2_w'] + p['pfc2_b']
    pe = relu(pro_protein @ p['pe1_w'] + p['pe1_b'])
    pe = relu(pe @ p['pe2_w'] + p['pe2_b'])
    de = relu(drug_smiles @ p['de1_w'] + p['de1_b'])
    de = relu(de @ p['de2_w'] + p['de2_b'])
    xc = jnp.concatenate([x, de, px, pe], axis=1)
    att = _mha(xc, p['wq'], p['wk'], p['wv'], p['wo'])
    xc = jnp.concatenate([xc, att], axis=1)
    xc = relu(xc @ p['fc1_w'] + p['fc1_b'])
    xc = relu(xc @ p['fc2_w'] + p['fc2_b'])
    return xc @ p['out_w'] + p['out_b']
